# baseline (speedup 1.0000x reference)
import functools

import jax
import jax.numpy as jnp
from jax import lax
from jax.experimental import pallas as pl
from jax.experimental.pallas import tpu as pltpu
from jax.experimental.pallas import tpu_sc as plsc

N_CELLS = 10000
N_GENES = 10000
D_FEAT = 128
N_EDGES = 320000

_NC = 2
_NS = 16
_NW = _NC * _NS
_LANES = 16

_EPW = N_EDGES // _NW
_CHUNK = 80
_NCHUNK = _EPW // _CHUNK
_NBLK = D_FEAT // _LANES


def _dot_decoder_body(src_hbm, dst_hbm, u_hbm, v_hbm, out_hbm,
                      sidx_all, didx_all, u0, v0, u1, v1, parts, out_v,
                      sem_u0, sem_v0, sem_u1, sem_v1):
    wid = lax.axis_index("s") * _NC + lax.axis_index("c")
    base = wid * _EPW
    lanes = lax.iota(jnp.int32, _LANES)

    pltpu.sync_copy(src_hbm.at[pl.ds(base, _EPW)], sidx_all)
    pltpu.sync_copy(dst_hbm.at[pl.ds(base, _EPW)], didx_all)

    bufs = ((u0, v0, sem_u0, sem_v0), (u1, v1, sem_u1, sem_v1))

    def issue(c, b):
        ub, vb, su, sv = bufs[b]
        pltpu.async_copy(u_hbm.at[sidx_all.at[pl.ds(c * _CHUNK, _CHUNK)]],
                         ub, su)
        pltpu.async_copy(v_hbm.at[didx_all.at[pl.ds(c * _CHUNK, _CHUNK)]],
                         vb, sv)

    def wait(b):
        ub, vb, su, sv = bufs[b]
        pltpu.make_async_copy(u_hbm.at[pl.ds(0, _CHUNK)], ub, su).wait()
        pltpu.make_async_copy(v_hbm.at[pl.ds(0, _CHUNK)], vb, sv).wait()


    def compute(c, b):
        ub, vb, _, _ = bufs[b]

        def edge_body(e2, _):
            for t in range(2):
                e = e2 * 2 + t
                acc0 = None
                acc1 = None
                for j in range(_NBLK // 2):
                    uab = plsc.bitcast(ub[e, pl.ds(j * _LANES, _LANES)],
                                       jnp.bfloat16)
                    vab = plsc.bitcast(vb[e, pl.ds(j * _LANES, _LANES)],
                                       jnp.bfloat16)
                    ue, uo = plsc.unpack(uab, format=plsc.PackFormat.INTERLEAVED,
                                         preferred_element_type=jnp.float32)
                    ve, vo = plsc.unpack(vab, format=plsc.PackFormat.INTERLEAVED,
                                         preferred_element_type=jnp.float32)
                    if acc0 is None:
                        acc0 = ue * ve
                        acc1 = uo * vo
                    else:
                        acc0 = acc0 + ue * ve
                        acc1 = acc1 + uo * vo
                parts[e, :] = plsc.cumsum(acc0 + acc1)
            return _

        lax.fori_loop(0, _CHUNK // 2, edge_body, None)

        def group_body(g, _):
            rows = g * _LANES + lanes
            x = plsc.load_gather(
                parts, [rows, jnp.full((_LANES,), _LANES - 1, jnp.int32)])
            pred = 1.0 / (1.0 + jnp.exp(-x))
            out_v[pl.ds(c * _CHUNK + g * _LANES, _LANES)] = pred
            return _

        lax.fori_loop(0, _CHUNK // _LANES, group_body, None)

    issue(0, 0)

    def pair_body(c2, _):
        c = c2 * 2
        wait(0)
        issue(c + 1, 1)
        compute(c, 0)
        wait(1)
        issue(c + 2, 0)
        compute(c + 1, 1)
        return _

    lax.fori_loop(0, (_NCHUNK - 1) // 2, pair_body, None)
    wait(0)
    compute(_NCHUNK - 1, 0)

    pltpu.sync_copy(out_v, out_hbm.at[pl.ds(base, _EPW)])


@jax.jit
def _dot_decoder(src, dst, ufeats, ifeats):
    mesh = plsc.VectorSubcoreMesh(core_axis_name="c", subcore_axis_name="s")
    scores = pl.kernel(
        _dot_decoder_body,
        out_type=jax.ShapeDtypeStruct((N_EDGES,), jnp.float32),
        mesh=mesh,
        compiler_params=pltpu.CompilerParams(needs_layout_passes=False,
                                             use_tc_tiling_on_sc=False),
        scratch_types=[
            pltpu.VMEM((_EPW,), jnp.int32),
            pltpu.VMEM((_EPW,), jnp.int32),
            pltpu.VMEM((_CHUNK, D_FEAT // 2), jnp.uint32),
            pltpu.VMEM((_CHUNK, D_FEAT // 2), jnp.uint32),
            pltpu.VMEM((_CHUNK, D_FEAT // 2), jnp.uint32),
            pltpu.VMEM((_CHUNK, D_FEAT // 2), jnp.uint32),
            pltpu.VMEM((_CHUNK, _LANES), jnp.float32),
            pltpu.VMEM((_EPW,), jnp.float32),
            pltpu.SemaphoreType.DMA,
            pltpu.SemaphoreType.DMA,
            pltpu.SemaphoreType.DMA,
            pltpu.SemaphoreType.DMA,
        ],
    )(src, dst, ufeats, ifeats)
    return scores


def kernel(ufeats, ifeats, edge_index):
    src = edge_index[0].astype(jnp.int32)
    dst = edge_index[1].astype(jnp.int32)
    scores = _dot_decoder(src, dst, _pack_bf16(ufeats), _pack_bf16(ifeats))
    return scores.reshape(N_EDGES, 1)


def _pack_bf16(x):
    b = lax.bitcast_convert_type(x, jnp.uint32)
    r = (b + 0x7FFF + ((b >> 16) & 1)) >> 16
    lo = r[:, 0::2]
    hi = r[:, 1::2]
    return (lo | (hi << 16)).astype(jnp.uint32)


# speedup vs baseline: 1.0679x; 1.0679x over previous
import functools

import jax
import jax.numpy as jnp
from jax import lax
from jax.experimental import pallas as pl
from jax.experimental.pallas import tpu as pltpu
from jax.experimental.pallas import tpu_sc as plsc

N_CELLS = 10000
N_GENES = 10000
D_FEAT = 128
N_EDGES = 320000

_NC = 2
_NS = 16
_NW = _NC * _NS
_LANES = 16

_EPW = N_EDGES // _NW
_CHUNK = 80
_NCHUNK = _EPW // _CHUNK
_NBLK = D_FEAT // _LANES


def _dot_decoder_body(src_hbm, dst_hbm, u_hbm, v_hbm, out_hbm,
                      sidx_all, didx_all, u0, v0, u1, v1, parts, out_v,
                      sem_u0, sem_v0, sem_u1, sem_v1):
    wid = lax.axis_index("s") * _NC + lax.axis_index("c")
    base = wid * _EPW
    lanes = lax.iota(jnp.int32, _LANES)

    pltpu.sync_copy(src_hbm.at[pl.ds(base, _EPW)], sidx_all)
    pltpu.sync_copy(dst_hbm.at[pl.ds(base, _EPW)], didx_all)

    bufs = ((u0, v0, sem_u0, sem_v0), (u1, v1, sem_u1, sem_v1))

    def issue(c, b):
        ub, vb, su, sv = bufs[b]
        pltpu.async_copy(u_hbm.at[sidx_all.at[pl.ds(c * _CHUNK, _CHUNK)]],
                         ub, su)
        pltpu.async_copy(v_hbm.at[didx_all.at[pl.ds(c * _CHUNK, _CHUNK)]],
                         vb, sv)

    def wait(b):
        ub, vb, su, sv = bufs[b]
        pltpu.make_async_copy(u_hbm.at[pl.ds(0, _CHUNK)], ub, su).wait()
        pltpu.make_async_copy(v_hbm.at[pl.ds(0, _CHUNK)], vb, sv).wait()


    def compute(c, b):
        ub, vb, _, _ = bufs[b]

        def edge_body(e2, _):
            for t in range(2):
                e = e2 * 2 + t
                acc0 = None
                acc1 = None
                for j in range(_NBLK // 2):
                    uab = plsc.bitcast(ub[e, pl.ds(j * _LANES, _LANES)],
                                       jnp.bfloat16)
                    vab = plsc.bitcast(vb[e, pl.ds(j * _LANES, _LANES)],
                                       jnp.bfloat16)
                    ue, uo = plsc.unpack(uab, format=plsc.PackFormat.INTERLEAVED,
                                         preferred_element_type=jnp.float32)
                    ve, vo = plsc.unpack(vab, format=plsc.PackFormat.INTERLEAVED,
                                         preferred_element_type=jnp.float32)
                    if acc0 is None:
                        acc0 = ue * ve
                        acc1 = uo * vo
                    else:
                        acc0 = acc0 + ue * ve
                        acc1 = acc1 + uo * vo
                parts[e, :] = acc0 + acc1
            return _

        lax.fori_loop(0, _CHUNK // 2, edge_body, None)

        def group_body(g, _):
            rows = g * _LANES + lanes
            acc0 = plsc.load_gather(parts, [rows, jnp.zeros((_LANES,), jnp.int32)])
            acc1 = plsc.load_gather(parts, [rows, jnp.full((_LANES,), 1, jnp.int32)])
            for k in range(2, _LANES, 2):
                acc0 = acc0 + plsc.load_gather(
                    parts, [rows, jnp.full((_LANES,), k, jnp.int32)])
                acc1 = acc1 + plsc.load_gather(
                    parts, [rows, jnp.full((_LANES,), k + 1, jnp.int32)])
            x = acc0 + acc1
            pred = 1.0 / (1.0 + jnp.exp(-x))
            out_v[pl.ds(c * _CHUNK + g * _LANES, _LANES)] = pred
            return _

        lax.fori_loop(0, _CHUNK // _LANES, group_body, None)

    issue(0, 0)

    def pair_body(c2, _):
        c = c2 * 2
        wait(0)
        issue(c + 1, 1)
        compute(c, 0)
        wait(1)
        issue(c + 2, 0)
        compute(c + 1, 1)
        return _

    lax.fori_loop(0, (_NCHUNK - 1) // 2, pair_body, None)
    wait(0)
    compute(_NCHUNK - 1, 0)

    pltpu.sync_copy(out_v, out_hbm.at[pl.ds(base, _EPW)])


@jax.jit
def _dot_decoder(src, dst, ufeats, ifeats):
    mesh = plsc.VectorSubcoreMesh(core_axis_name="c", subcore_axis_name="s")
    scores = pl.kernel(
        _dot_decoder_body,
        out_type=jax.ShapeDtypeStruct((N_EDGES,), jnp.float32),
        mesh=mesh,
        compiler_params=pltpu.CompilerParams(needs_layout_passes=False,
                                             use_tc_tiling_on_sc=False),
        scratch_types=[
            pltpu.VMEM((_EPW,), jnp.int32),
            pltpu.VMEM((_EPW,), jnp.int32),
            pltpu.VMEM((_CHUNK, D_FEAT // 2), jnp.uint32),
            pltpu.VMEM((_CHUNK, D_FEAT // 2), jnp.uint32),
            pltpu.VMEM((_CHUNK, D_FEAT // 2), jnp.uint32),
            pltpu.VMEM((_CHUNK, D_FEAT // 2), jnp.uint32),
            pltpu.VMEM((_CHUNK, _LANES), jnp.float32),
            pltpu.VMEM((_EPW,), jnp.float32),
            pltpu.SemaphoreType.DMA,
            pltpu.SemaphoreType.DMA,
            pltpu.SemaphoreType.DMA,
            pltpu.SemaphoreType.DMA,
        ],
    )(src, dst, ufeats, ifeats)
    return scores


def kernel(ufeats, ifeats, edge_index):
    src = edge_index[0].astype(jnp.int32)
    dst = edge_index[1].astype(jnp.int32)
    scores = _dot_decoder(src, dst, _pack_bf16(ufeats), _pack_bf16(ifeats))
    return scores.reshape(N_EDGES, 1)


def _pack_bf16(x):
    b = lax.bitcast_convert_type(x, jnp.uint32)
    r = (b + 0x7FFF + ((b >> 16) & 1)) >> 16
    lo = r[:, 0::2]
    hi = r[:, 1::2]
    return (lo | (hi << 16)).astype(jnp.uint32)


# speedup vs baseline: 3.0144x; 2.8228x over previous
import functools

import jax
import jax.numpy as jnp
from jax import lax
from jax.experimental import pallas as pl
from jax.experimental.pallas import tpu as pltpu
from jax.experimental.pallas import tpu_sc as plsc

N_CELLS = 10000
N_GENES = 10000
D_FEAT = 128
N_EDGES = 320000

_NC = 2
_NS = 16
_NW = _NC * _NS
_LANES = 16

_EPW = N_EDGES // _NW
_CHUNK = 80
_NCHUNK = _EPW // _CHUNK
_NBLK = D_FEAT // _LANES


def _dot_decoder_body(src_hbm, dst_hbm, u_hbm, v_hbm, out_hbm,
                      sidx_all, didx_all, u0, v0, u1, v1, parts, out_v,
                      sem_u0, sem_v0, sem_u1, sem_v1):
    wid = lax.axis_index("s") * _NC + lax.axis_index("c")
    base = wid * _EPW
    lanes = lax.iota(jnp.int32, _LANES)

    pltpu.sync_copy(src_hbm.at[pl.ds(base, _EPW)], sidx_all)
    pltpu.sync_copy(dst_hbm.at[pl.ds(base, _EPW)], didx_all)

    bufs = ((u0, v0, sem_u0, sem_v0), (u1, v1, sem_u1, sem_v1))

    def issue(c, b):
        ub, vb, su, sv = bufs[b]
        pltpu.async_copy(u_hbm.at[sidx_all.at[pl.ds(c * _CHUNK, _CHUNK)]],
                         ub, su)
        pltpu.async_copy(v_hbm.at[didx_all.at[pl.ds(c * _CHUNK, _CHUNK)]],
                         vb, sv)

    def wait(b):
        ub, vb, su, sv = bufs[b]
        pltpu.make_async_copy(u_hbm.at[pl.ds(0, _CHUNK)], ub, su).wait()
        pltpu.make_async_copy(v_hbm.at[pl.ds(0, _CHUNK)], vb, sv).wait()


    def compute(c, b):
        ub, vb, _, _ = bufs[b]

        def edge_body(e2, _):
            for t in range(2):
                e = e2 * 2 + t
                acc0 = None
                acc1 = None
                for j in range(_NBLK // 2):
                    uab = plsc.bitcast(ub[e, pl.ds(j * _LANES, _LANES)],
                                       jnp.bfloat16)
                    vab = plsc.bitcast(vb[e, pl.ds(j * _LANES, _LANES)],
                                       jnp.bfloat16)
                    ue, uo = plsc.unpack(uab, format=plsc.PackFormat.INTERLEAVED,
                                         preferred_element_type=jnp.float32)
                    ve, vo = plsc.unpack(vab, format=plsc.PackFormat.INTERLEAVED,
                                         preferred_element_type=jnp.float32)
                    if acc0 is None:
                        acc0 = ue * ve
                        acc1 = uo * vo
                    else:
                        acc0 = acc0 + ue * ve
                        acc1 = acc1 + uo * vo
                parts[e, :] = acc0 + acc1
            return _

        lax.fori_loop(0, _CHUNK // 2, edge_body, None)

        def group_body(g, _):
            rows = g * _LANES + lanes
            acc0 = plsc.load_gather(parts, [rows, jnp.zeros((_LANES,), jnp.int32)])
            acc1 = plsc.load_gather(parts, [rows, jnp.full((_LANES,), 1, jnp.int32)])
            for k in range(2, _LANES, 2):
                acc0 = acc0 + plsc.load_gather(
                    parts, [rows, jnp.full((_LANES,), k, jnp.int32)])
                acc1 = acc1 + plsc.load_gather(
                    parts, [rows, jnp.full((_LANES,), k + 1, jnp.int32)])
            x = acc0 + acc1
            pred = 1.0 / (1.0 + jnp.exp(-x))
            out_v[pl.ds(c * _CHUNK + g * _LANES, _LANES)] = pred
            return _

        lax.fori_loop(0, _CHUNK // _LANES, group_body, None)

    issue(0, 0)

    def pair_body(c2, _):
        c = c2 * 2
        wait(0)
        issue(c + 1, 1)
        compute(c, 0)
        wait(1)
        issue(c + 2, 0)
        compute(c + 1, 1)
        return _

    lax.fori_loop(0, (_NCHUNK - 1) // 2, pair_body, None)
    wait(0)
    compute(_NCHUNK - 1, 0)

    pltpu.sync_copy(out_v, out_hbm.at[pl.ds(base, _EPW)])


@jax.jit
def _dot_decoder(src, dst, ufeats, ifeats):
    mesh = plsc.VectorSubcoreMesh(core_axis_name="c", subcore_axis_name="s")
    scores = pl.kernel(
        _dot_decoder_body,
        out_type=jax.ShapeDtypeStruct((N_EDGES,), jnp.float32),
        mesh=mesh,
        compiler_params=pltpu.CompilerParams(needs_layout_passes=False,
                                             use_tc_tiling_on_sc=False),
        scratch_types=[
            pltpu.VMEM((_EPW,), jnp.int32),
            pltpu.VMEM((_EPW,), jnp.int32),
            pltpu.VMEM((_CHUNK, D_FEAT // 2), jnp.uint32),
            pltpu.VMEM((_CHUNK, D_FEAT // 2), jnp.uint32),
            pltpu.VMEM((_CHUNK, D_FEAT // 2), jnp.uint32),
            pltpu.VMEM((_CHUNK, D_FEAT // 2), jnp.uint32),
            pltpu.VMEM((_CHUNK, _LANES), jnp.float32),
            pltpu.VMEM((_EPW,), jnp.float32),
            pltpu.SemaphoreType.DMA,
            pltpu.SemaphoreType.DMA,
            pltpu.SemaphoreType.DMA,
            pltpu.SemaphoreType.DMA,
        ],
    )(src, dst, ufeats, ifeats)
    return scores


def kernel(ufeats, ifeats, edge_index):
    src = edge_index[0].astype(jnp.int32)
    dst = edge_index[1].astype(jnp.int32)
    scores = _dot_decoder(src, dst, _pack_bf16(ufeats), _pack_bf16(ifeats))
    return scores.reshape(N_EDGES, 1)


def _pack_bf16(x):
    b = lax.bitcast_convert_type(x, jnp.uint32)
    r = (b + 0x7FFF + ((b >> 16) & 1)) >> 16
    lo = r[:, :D_FEAT // 2]
    hi = r[:, D_FEAT // 2:]
    return lo | (hi << 16)


# speedup vs baseline: 3.0454x; 1.0103x over previous
import functools

import jax
import jax.numpy as jnp
from jax import lax
from jax.experimental import pallas as pl
from jax.experimental.pallas import tpu as pltpu
from jax.experimental.pallas import tpu_sc as plsc

N_CELLS = 10000
N_GENES = 10000
D_FEAT = 128
N_EDGES = 320000

_NC = 2
_NS = 16
_NW = _NC * _NS
_LANES = 16

_EPW = N_EDGES // _NW
_CHUNK = 80
_NCHUNK = _EPW // _CHUNK
_NBLK = D_FEAT // _LANES
_UNROLL = 8


def _dot_decoder_body(src_hbm, dst_hbm, u_hbm, v_hbm, out_hbm,
                      sidx_all, didx_all, u0, v0, u1, v1, parts, out_v,
                      sem_u0, sem_v0, sem_u1, sem_v1):
    wid = lax.axis_index("s") * _NC + lax.axis_index("c")
    base = wid * _EPW
    lanes = lax.iota(jnp.int32, _LANES)

    pltpu.sync_copy(src_hbm.at[pl.ds(base, _EPW)], sidx_all)
    pltpu.sync_copy(dst_hbm.at[pl.ds(base, _EPW)], didx_all)

    bufs = ((u0, v0, sem_u0, sem_v0), (u1, v1, sem_u1, sem_v1))

    def issue(c, b):
        ub, vb, su, sv = bufs[b]
        pltpu.async_copy(u_hbm.at[sidx_all.at[pl.ds(c * _CHUNK, _CHUNK)]],
                         ub, su)
        pltpu.async_copy(v_hbm.at[didx_all.at[pl.ds(c * _CHUNK, _CHUNK)]],
                         vb, sv)

    def wait(b):
        ub, vb, su, sv = bufs[b]
        pltpu.make_async_copy(u_hbm.at[pl.ds(0, _CHUNK)], ub, su).wait()
        pltpu.make_async_copy(v_hbm.at[pl.ds(0, _CHUNK)], vb, sv).wait()


    def compute(c, b):
        ub, vb, _, _ = bufs[b]

        def edge_body(e2, _):
            for t in range(_UNROLL):
                e = e2 * _UNROLL + t
                acc0 = None
                acc1 = None
                for j in range(_NBLK // 2):
                    uab = plsc.bitcast(ub[e, pl.ds(j * _LANES, _LANES)],
                                       jnp.bfloat16)
                    vab = plsc.bitcast(vb[e, pl.ds(j * _LANES, _LANES)],
                                       jnp.bfloat16)
                    ue, uo = plsc.unpack(uab, format=plsc.PackFormat.INTERLEAVED,
                                         preferred_element_type=jnp.float32)
                    ve, vo = plsc.unpack(vab, format=plsc.PackFormat.INTERLEAVED,
                                         preferred_element_type=jnp.float32)
                    if acc0 is None:
                        acc0 = ue * ve
                        acc1 = uo * vo
                    else:
                        acc0 = acc0 + ue * ve
                        acc1 = acc1 + uo * vo
                parts[e, :] = acc0 + acc1
            return _

        lax.fori_loop(0, _CHUNK // _UNROLL, edge_body, None)

        for g in range(_CHUNK // _LANES):
            rows = g * _LANES + lanes
            acc0 = plsc.load_gather(parts, [rows, jnp.zeros((_LANES,), jnp.int32)])
            acc1 = plsc.load_gather(parts, [rows, jnp.full((_LANES,), 1, jnp.int32)])
            for k in range(2, _LANES, 2):
                acc0 = acc0 + plsc.load_gather(
                    parts, [rows, jnp.full((_LANES,), k, jnp.int32)])
                acc1 = acc1 + plsc.load_gather(
                    parts, [rows, jnp.full((_LANES,), k + 1, jnp.int32)])
            x = acc0 + acc1
            pred = 1.0 / (1.0 + jnp.exp(-x))
            out_v[pl.ds(c * _CHUNK + g * _LANES, _LANES)] = pred

    issue(0, 0)

    def pair_body(c2, _):
        c = c2 * 2
        wait(0)
        issue(c + 1, 1)
        compute(c, 0)
        wait(1)
        issue(c + 2, 0)
        compute(c + 1, 1)
        return _

    lax.fori_loop(0, (_NCHUNK - 1) // 2, pair_body, None)
    wait(0)
    compute(_NCHUNK - 1, 0)

    pltpu.sync_copy(out_v, out_hbm.at[pl.ds(base, _EPW)])


@jax.jit
def _dot_decoder(src, dst, ufeats, ifeats):
    mesh = plsc.VectorSubcoreMesh(core_axis_name="c", subcore_axis_name="s")
    scores = pl.kernel(
        _dot_decoder_body,
        out_type=jax.ShapeDtypeStruct((N_EDGES,), jnp.float32),
        mesh=mesh,
        compiler_params=pltpu.CompilerParams(needs_layout_passes=False,
                                             use_tc_tiling_on_sc=False),
        scratch_types=[
            pltpu.VMEM((_EPW,), jnp.int32),
            pltpu.VMEM((_EPW,), jnp.int32),
            pltpu.VMEM((_CHUNK, D_FEAT // 2), jnp.uint32),
            pltpu.VMEM((_CHUNK, D_FEAT // 2), jnp.uint32),
            pltpu.VMEM((_CHUNK, D_FEAT // 2), jnp.uint32),
            pltpu.VMEM((_CHUNK, D_FEAT // 2), jnp.uint32),
            pltpu.VMEM((_CHUNK, _LANES), jnp.float32),
            pltpu.VMEM((_EPW,), jnp.float32),
            pltpu.SemaphoreType.DMA,
            pltpu.SemaphoreType.DMA,
            pltpu.SemaphoreType.DMA,
            pltpu.SemaphoreType.DMA,
        ],
    )(src, dst, ufeats, ifeats)
    return scores


def kernel(ufeats, ifeats, edge_index):
    src = edge_index[0].astype(jnp.int32)
    dst = edge_index[1].astype(jnp.int32)
    scores = _dot_decoder(src, dst, _pack_bf16(ufeats), _pack_bf16(ifeats))
    return scores.reshape(N_EDGES, 1)


def _pack_bf16(x):
    b = lax.bitcast_convert_type(x, jnp.uint32)
    r = (b + 0x7FFF + ((b >> 16) & 1)) >> 16
    lo = r[:, :D_FEAT // 2]
    hi = r[:, D_FEAT // 2:]
    return lo | (hi << 16)


# speedup vs baseline: 3.0632x; 1.0059x over previous
import functools

import jax
import jax.numpy as jnp
from jax import lax
from jax.experimental import pallas as pl
from jax.experimental.pallas import tpu as pltpu
from jax.experimental.pallas import tpu_sc as plsc

N_CELLS = 10000
N_GENES = 10000
D_FEAT = 128
N_EDGES = 320000

_NC = 2
_NS = 16
_NW = _NC * _NS
_LANES = 16

_EPW = N_EDGES // _NW
_CHUNK = 80
_NCHUNK = _EPW // _CHUNK
_NBLK = D_FEAT // _LANES
_UNROLL = 8


def _dot_decoder_body(src_hbm, dst_hbm, u_hbm, v_hbm, out_hbm,
                      sidx_all, didx_all, u0, v0, u1, v1, parts, out_v,
                      sem_u0, sem_v0, sem_u1, sem_v1):
    wid = lax.axis_index("s") * _NC + lax.axis_index("c")
    base = wid * _EPW
    lanes = lax.iota(jnp.int32, _LANES)

    pltpu.sync_copy(src_hbm.at[pl.ds(base, _EPW)], sidx_all)
    pltpu.sync_copy(dst_hbm.at[pl.ds(base, _EPW)], didx_all)

    bufs = ((u0, v0, sem_u0, sem_v0), (u1, v1, sem_u1, sem_v1))

    def issue(c, b):
        ub, vb, su, sv = bufs[b]
        pltpu.async_copy(u_hbm.at[sidx_all.at[pl.ds(c * _CHUNK, _CHUNK)]],
                         ub, su)
        pltpu.async_copy(v_hbm.at[didx_all.at[pl.ds(c * _CHUNK, _CHUNK)]],
                         vb, sv)

    def wait(b):
        ub, vb, su, sv = bufs[b]
        pltpu.make_async_copy(u_hbm.at[pl.ds(0, _CHUNK)], ub, su).wait()
        pltpu.make_async_copy(v_hbm.at[pl.ds(0, _CHUNK)], vb, sv).wait()


    def compute(c, b):
        ub, vb, _, _ = bufs[b]

        def edge_body(e2, _):
            for t in range(_UNROLL):
                e = e2 * _UNROLL + t
                acc0 = None
                acc1 = None
                for j in range(_NBLK // 2):
                    uab = plsc.bitcast(ub[e, pl.ds(j * _LANES, _LANES)],
                                       jnp.bfloat16)
                    vab = plsc.bitcast(vb[e, pl.ds(j * _LANES, _LANES)],
                                       jnp.bfloat16)
                    prod = uab * vab
                    pe, po = plsc.unpack(prod,
                                         format=plsc.PackFormat.INTERLEAVED,
                                         preferred_element_type=jnp.float32)
                    if acc0 is None:
                        acc0 = pe
                        acc1 = po
                    else:
                        acc0 = acc0 + pe
                        acc1 = acc1 + po
                parts[e, :] = acc0 + acc1
            return _

        lax.fori_loop(0, _CHUNK // _UNROLL, edge_body, None)

        for g in range(_CHUNK // _LANES):
            rows = g * _LANES + lanes
            acc0 = plsc.load_gather(parts, [rows, jnp.zeros((_LANES,), jnp.int32)])
            acc1 = plsc.load_gather(parts, [rows, jnp.full((_LANES,), 1, jnp.int32)])
            for k in range(2, _LANES, 2):
                acc0 = acc0 + plsc.load_gather(
                    parts, [rows, jnp.full((_LANES,), k, jnp.int32)])
                acc1 = acc1 + plsc.load_gather(
                    parts, [rows, jnp.full((_LANES,), k + 1, jnp.int32)])
            x = acc0 + acc1
            pred = 1.0 / (1.0 + jnp.exp(-x))
            out_v[pl.ds(c * _CHUNK + g * _LANES, _LANES)] = pred

    issue(0, 0)

    def pair_body(c2, _):
        c = c2 * 2
        wait(0)
        issue(c + 1, 1)
        compute(c, 0)
        wait(1)
        issue(c + 2, 0)
        compute(c + 1, 1)
        return _

    lax.fori_loop(0, (_NCHUNK - 1) // 2, pair_body, None)
    wait(0)
    compute(_NCHUNK - 1, 0)

    pltpu.sync_copy(out_v, out_hbm.at[pl.ds(base, _EPW)])


@jax.jit
def _dot_decoder(src, dst, ufeats, ifeats):
    mesh = plsc.VectorSubcoreMesh(core_axis_name="c", subcore_axis_name="s")
    scores = pl.kernel(
        _dot_decoder_body,
        out_type=jax.ShapeDtypeStruct((N_EDGES,), jnp.float32),
        mesh=mesh,
        compiler_params=pltpu.CompilerParams(needs_layout_passes=False,
                                             use_tc_tiling_on_sc=False),
        scratch_types=[
            pltpu.VMEM((_EPW,), jnp.int32),
            pltpu.VMEM((_EPW,), jnp.int32),
            pltpu.VMEM((_CHUNK, D_FEAT // 2), jnp.uint32),
            pltpu.VMEM((_CHUNK, D_FEAT // 2), jnp.uint32),
            pltpu.VMEM((_CHUNK, D_FEAT // 2), jnp.uint32),
            pltpu.VMEM((_CHUNK, D_FEAT // 2), jnp.uint32),
            pltpu.VMEM((_CHUNK, _LANES), jnp.float32),
            pltpu.VMEM((_EPW,), jnp.float32),
            pltpu.SemaphoreType.DMA,
            pltpu.SemaphoreType.DMA,
            pltpu.SemaphoreType.DMA,
            pltpu.SemaphoreType.DMA,
        ],
    )(src, dst, ufeats, ifeats)
    return scores


def kernel(ufeats, ifeats, edge_index):
    src = edge_index[0].astype(jnp.int32)
    dst = edge_index[1].astype(jnp.int32)
    scores = _dot_decoder(src, dst, _pack_bf16(ufeats), _pack_bf16(ifeats))
    return scores.reshape(N_EDGES, 1)


def _pack_bf16(x):
    b = lax.bitcast_convert_type(x, jnp.uint32)
    r = (b + 0x7FFF + ((b >> 16) & 1)) >> 16
    lo = r[:, :D_FEAT // 2]
    hi = r[:, D_FEAT // 2:]
    return lo | (hi << 16)
